# Initial kernel scaffold; baseline (speedup 1.0000x reference)
import jax
import jax.numpy as jnp
from jax.experimental import pallas as pl


def kernel(x, edge_index, edge_attr, Wl1, bl1, Wr1, br1, We1, att1, bias1, Wskip, bskip, gamma, beta, Wl2, bl2, Wr2, br2, We2, att2, bias2, Wc1, bc1, Wt1, bt1, Wc2, bc2, Wt2, bt2):
    raise NotImplementedError("write your pallas kernel here")


# speedup vs baseline: 32.4398x; 32.4398x over previous
import functools
import jax
import jax.numpy as jnp
from jax import lax
from jax.experimental import pallas as pl
from jax.experimental.pallas import tpu as pltpu
from jax.experimental.pallas import tpu_sc as plsc

_N = 50000
_E = 800000
_H = 8
_D = 16
_HD = 128

_NC = 2
_NS = 16
_NW = _NC * _NS


def _sc_gather(table, idx):
    e = idx.shape[0]
    d = table.shape[1]
    per_w = e // _NW
    ch = 1000
    n_ch = per_w // ch
    mesh = plsc.VectorSubcoreMesh(core_axis_name="c", subcore_axis_name="s")

    @functools.partial(
        pl.kernel, mesh=mesh,
        out_type=jax.ShapeDtypeStruct((e, d), jnp.float32),
        scratch_types=[
            pltpu.VMEM((ch,), jnp.int32),
            pltpu.VMEM((ch, d), jnp.float32),
            pltpu.SemaphoreType.DMA,
        ],
        compiler_params=pltpu.CompilerParams(use_tc_tiling_on_sc=False),
    )
    def k(table_hbm, idx_hbm, out_hbm, idx_v, rows_v, sem):
        wid = lax.axis_index("s") * _NC + lax.axis_index("c")

        def body(i, carry):
            base = wid * per_w + i * ch
            pltpu.sync_copy(idx_hbm.at[pl.ds(base, ch)], idx_v)
            pltpu.async_copy(table_hbm.at[idx_v], rows_v, sem).wait()
            pltpu.sync_copy(rows_v, out_hbm.at[pl.ds(base, ch)])
            return carry

        lax.fori_loop(0, n_ch, body, 0)

    return k(table, idx)


def _sc_scatter_add(vals, idx, zeros):
    e, d = vals.shape
    n = zeros.shape[0]
    per_w = e // _NW
    ch = 200
    n_ch = per_w // ch
    rows_pw = n // _NS
    mesh = plsc.VectorSubcoreMesh(core_axis_name="c", subcore_axis_name="s")

    @functools.partial(
        pl.kernel, mesh=mesh,
        out_type=jax.ShapeDtypeStruct((_NC, n, d), jnp.float32),
        scratch_types=[
            pltpu.VMEM((ch,), jnp.int32),
            pltpu.VMEM((ch, d), jnp.float32),
            pltpu.VMEM_SHARED((n, d), jnp.float32),
        ],
        compiler_params=pltpu.CompilerParams(use_tc_tiling_on_sc=False),
    )
    def k(vals_hbm, idx_hbm, zeros_hbm, out_hbm, idx_v, vals_v, acc_sh):
        cid = lax.axis_index("c")
        sid = lax.axis_index("s")
        wid = sid * _NC + cid
        pltpu.sync_copy(zeros_hbm.at[pl.ds(sid * rows_pw, rows_pw)],
                        acc_sh.at[pl.ds(sid * rows_pw, rows_pw)])
        plsc.subcore_barrier()

        def body(i, carry):
            base = wid * per_w + i * ch
            pltpu.sync_copy(idx_hbm.at[pl.ds(base, ch)], idx_v)
            pltpu.sync_copy(vals_hbm.at[pl.ds(base, ch)], vals_v)
            pltpu.sync_copy(vals_v, acc_sh.at[idx_v], add=True)
            return carry

        lax.fori_loop(0, n_ch, body, 0)
        plsc.subcore_barrier()
        pltpu.sync_copy(acc_sh.at[pl.ds(sid * rows_pw, rows_pw)],
                        out_hbm.at[cid, pl.ds(sid * rows_pw, rows_pw)])

    return k(vals, idx, zeros)


_BE = 8000
_BN = 2000


def _edge1_body(xs_ref, xd_ref, ea_ref, wl0_ref, wl1_ref, wr0_ref, wr1_ref,
                bb_ref, we_ref, attf_ref, s_ref, out_ref):
    x0 = xs_ref[:, 0:1]
    x1 = xs_ref[:, 1:2]
    y0 = xd_ref[:, 0:1]
    y1 = xd_ref[:, 1:2]
    e = (x0 * wl0_ref[0:1, :] + x1 * wl1_ref[0:1, :]
         + y0 * wr0_ref[0:1, :] + y1 * wr1_ref[0:1, :]
         + ea_ref[:, 0:1] * we_ref[0:1, :] + bb_ref[0:1, :])
    e = jnp.where(e >= 0.0, e, 0.2 * e)
    alpha = jnp.dot(e * attf_ref[0:1, :], s_ref[:, :],
                    preferred_element_type=jnp.float32)
    ae = jnp.exp(alpha)
    out_ref[:, :] = jnp.concatenate(
        [ae, ae * x0, ae * x1, jnp.zeros_like(ae)], axis=1)


def _node1_body(a1a_ref, a1b_ref, x_ref, r0_ref, r1_ref, rb_ref,
                wsk_ref, bb_ref, hpre_ref, st_ref):
    i = pl.program_id(0)
    a = a1a_ref[:, :] + a1b_ref[:, :]
    den = a[:, 0:8] + 1e-16
    g0 = a[:, 8:16] / den
    g1 = a[:, 16:24] / den
    s = a[:, 0:8] / den
    h1 = (jnp.dot(g0, r0_ref[:, :], preferred_element_type=jnp.float32)
          + jnp.dot(g1, r1_ref[:, :], preferred_element_type=jnp.float32)
          + jnp.dot(s, rb_ref[:, :], preferred_element_type=jnp.float32))
    h = (h1 + x_ref[:, 0:1] * wsk_ref[0:1, :]
         + x_ref[:, 1:2] * wsk_ref[1:2, :] + bb_ref[0:1, :])
    hpre_ref[:, :] = h

    @pl.when(i == 0)
    def _():
        st_ref[:, :] = jnp.zeros_like(st_ref)

    st_ref[0:1, :] += jnp.sum(h, axis=0, keepdims=True)
    st_ref[1:2, :] += jnp.sum(h * h, axis=0, keepdims=True)


def _node1b_body(hpre_ref, st_ref, gam_ref, bet_ref, wl2_ref, wr2_ref,
                 b2_ref, xl2_ref, xr2_ref):
    mu = st_ref[0:1, :] * (1.0 / _N)
    var = st_ref[1:2, :] * (1.0 / _N) - mu * mu
    h = (hpre_ref[:, :] - mu) * jax.lax.rsqrt(var + 1e-5)
    h = h * gam_ref[0:1, :] + bet_ref[0:1, :]
    h = jnp.where(h > 0.0, h, jnp.exp(h) - 1.0)
    xl2_ref[:, :] = jnp.dot(h, wl2_ref[:, :],
                            preferred_element_type=jnp.float32) + b2_ref[0:1, 0:16]
    xr2_ref[:, :] = jnp.dot(h, wr2_ref[:, :],
                            preferred_element_type=jnp.float32) + b2_ref[0:1, 16:32]


def _edge2_body(xls_ref, xrd_ref, ea_ref, we2_ref, att2_ref, out_ref):
    e = xls_ref[:, :] + xrd_ref[:, :] + ea_ref[:, 0:1] * we2_ref[0:1, :]
    e = jnp.where(e >= 0.0, e, 0.2 * e)
    alpha = jnp.sum(e * att2_ref[0:1, :], axis=1, keepdims=True)
    ae = jnp.exp(alpha)
    z = jnp.zeros_like(xls_ref[:, 1:16])
    out_ref[:, :] = jnp.concatenate([ae, z, ae * xls_ref[:, :]], axis=1)


def _node2_body(a2a_ref, a2b_ref, b2_ref, wt1_ref, bt1_ref, wc1_ref, bc1_ref,
                wt2_ref, bt2_ref, wc2_ref, bc2_ref, out_ref, dn2_ref):
    a = a2a_ref[:, :] + a2b_ref[:, :]
    dn2_ref[:, :] = a[:, 0:16]
    h2 = a[:, 16:32] / (a[:, 0:1] + 1e-16) + b2_ref[0:1, :]
    h2 = jnp.where(h2 > 0.0, h2, jnp.exp(h2) - 1.0)
    ht = jnp.dot(h2, wt1_ref[:, :], preferred_element_type=jnp.float32) + bt1_ref[0:1, :]
    hc = jnp.dot(h2, wc1_ref[:, :], preferred_element_type=jnp.float32) + bc1_ref[0:1, :]
    ht = jnp.dot(ht, wt2_ref[:, :], preferred_element_type=jnp.float32) + bt2_ref[0:1, :]
    hc = jnp.dot(hc, wc2_ref[:, :], preferred_element_type=jnp.float32) + bc2_ref[0:1, :]
    out_ref[:, :] = jnp.concatenate(
        [hc[:, 0:4], ht[:, 0:6], jnp.zeros_like(h2[:, 0:6])], axis=1)


def _a2_body(p2_ref, g2_ref, out_ref):
    a2 = p2_ref[:, 0:1] / (g2_ref[:, 0:1] + 1e-16)
    out_ref[:, :] = jnp.concatenate([a2] * 8, axis=1)


def _row_spec(b, w):
    return pl.BlockSpec((b, w), lambda i: (i, 0))


def _full_spec(shape):
    return pl.BlockSpec(shape, lambda i: tuple(0 for _ in shape))


def kernel(x, edge_index, edge_attr, Wl1, bl1, Wr1, br1, We1, att1, bias1,
           Wskip, bskip, gamma, beta, Wl2, bl2, Wr2, br2, We2, att2, bias2,
           Wc1, bc1, Wt1, bt1, Wc2, bc2, Wt2, bt2):
    src = edge_index[0]
    dst = edge_index[1]

    sel = jnp.repeat(jnp.eye(_H, dtype=jnp.float32), _D, axis=0)
    attf = att1.reshape(1, _HD)
    r0 = sel.T * Wl1[0][None, :]
    r1 = sel.T * Wl1[1][None, :]
    rb = sel.T * bl1[None, :]
    bb1 = (bl1 + br1).reshape(1, _HD)
    b2cat = jnp.concatenate([bl2, br2]).reshape(1, 32)

    x_pad = jnp.pad(x, ((0, 0), (0, 14)))
    zeros32 = jnp.zeros((_N, 32), jnp.float32)

    xs = _sc_gather(x_pad, src)
    xd = _sc_gather(x_pad, dst)

    ge = _E // _BE
    p1 = pl.pallas_call(
        _edge1_body,
        grid=(ge,),
        in_specs=[_row_spec(_BE, 16), _row_spec(_BE, 16), _row_spec(_BE, 1),
                  _full_spec((1, _HD)), _full_spec((1, _HD)),
                  _full_spec((1, _HD)), _full_spec((1, _HD)),
                  _full_spec((1, _HD)), _full_spec((1, _HD)),
                  _full_spec((1, _HD)), _full_spec((_HD, _H))],
        out_specs=_row_spec(_BE, 32),
        out_shape=jax.ShapeDtypeStruct((_E, 32), jnp.float32),
    )(xs, xd, edge_attr, Wl1[0].reshape(1, -1), Wl1[1].reshape(1, -1),
      Wr1[0].reshape(1, -1), Wr1[1].reshape(1, -1), bb1,
      We1.reshape(1, -1), attf, sel)

    a1 = _sc_scatter_add(p1, dst, zeros32)

    gn = _N // _BN
    hpre, stats = pl.pallas_call(
        _node1_body,
        grid=(gn,),
        in_specs=[_row_spec(_BN, 32), _row_spec(_BN, 32), _row_spec(_BN, 2),
                  _full_spec((_H, _HD)), _full_spec((_H, _HD)),
                  _full_spec((_H, _HD)), _full_spec((2, _HD)),
                  _full_spec((1, _HD))],
        out_specs=[_row_spec(_BN, _HD), _full_spec((2, _HD))],
        out_shape=[jax.ShapeDtypeStruct((_N, _HD), jnp.float32),
                   jax.ShapeDtypeStruct((2, _HD), jnp.float32)],
    )(a1[0], a1[1], x, r0, r1, rb, Wskip,
      (bias1 + bskip).reshape(1, -1))

    xl2, xr2 = pl.pallas_call(
        _node1b_body,
        grid=(gn,),
        in_specs=[_row_spec(_BN, _HD), _full_spec((2, _HD)),
                  _full_spec((1, _HD)), _full_spec((1, _HD)),
                  _full_spec((_HD, _D)), _full_spec((_HD, _D)),
                  _full_spec((1, 32))],
        out_specs=[_row_spec(_BN, _D), _row_spec(_BN, _D)],
        out_shape=[jax.ShapeDtypeStruct((_N, _D), jnp.float32),
                   jax.ShapeDtypeStruct((_N, _D), jnp.float32)],
    )(hpre, stats, gamma.reshape(1, -1), beta.reshape(1, -1), Wl2, Wr2, b2cat)

    xls = _sc_gather(xl2, src)
    xrd = _sc_gather(xr2, dst)

    p2 = pl.pallas_call(
        _edge2_body,
        grid=(ge,),
        in_specs=[_row_spec(_BE, 16), _row_spec(_BE, 16), _row_spec(_BE, 1),
                  _full_spec((1, _D)), _full_spec((1, _D))],
        out_specs=_row_spec(_BE, 32),
        out_shape=jax.ShapeDtypeStruct((_E, 32), jnp.float32),
    )(xls, xrd, edge_attr, We2.reshape(1, -1), att2.reshape(1, -1))

    a2t = _sc_scatter_add(p2, dst, zeros32)

    out16, dn2 = pl.pallas_call(
        _node2_body,
        grid=(gn,),
        in_specs=[_row_spec(_BN, 32), _row_spec(_BN, 32),
                  _full_spec((1, _D)),
                  _full_spec((_D, _D)), _full_spec((1, _D)),
                  _full_spec((_D, _D)), _full_spec((1, _D)),
                  _full_spec((_D, 6)), _full_spec((1, 6)),
                  _full_spec((_D, 4)), _full_spec((1, 4))],
        out_specs=[_row_spec(_BN, 16), _row_spec(_BN, 16)],
        out_shape=[jax.ShapeDtypeStruct((_N, 16), jnp.float32),
                   jax.ShapeDtypeStruct((_N, 16), jnp.float32)],
    )(a2t[0], a2t[1], bias2.reshape(1, -1), Wt1, bt1.reshape(1, -1),
      Wc1, bc1.reshape(1, -1), Wt2, bt2.reshape(1, -1), Wc2, bc2.reshape(1, -1))

    g2 = _sc_gather(dn2, dst)

    a2v = pl.pallas_call(
        _a2_body,
        grid=(ge,),
        in_specs=[_row_spec(_BE, 32), _row_spec(_BE, 16)],
        out_specs=_row_spec(_BE, 8),
        out_shape=jax.ShapeDtypeStruct((_E, 8), jnp.float32),
    )(p2, g2)

    return out16[:, 0:10], a2v[:, 0:1]


# speedup vs baseline: 32.6841x; 1.0075x over previous
import functools
import jax
import jax.numpy as jnp
from jax import lax
from jax.experimental import pallas as pl
from jax.experimental.pallas import tpu as pltpu
from jax.experimental.pallas import tpu_sc as plsc

_N = 50000
_E = 800000
_H = 8
_D = 16
_HD = 128

_NC = 2
_NS = 16
_NW = _NC * _NS


def _sc_gather(table, idx):
    e = idx.shape[0]
    d = table.shape[1]
    per_w = e // _NW
    ch = 5000
    n_ch = per_w // ch
    mesh = plsc.VectorSubcoreMesh(core_axis_name="c", subcore_axis_name="s")

    @functools.partial(
        pl.kernel, mesh=mesh,
        out_type=jax.ShapeDtypeStruct((e, d), jnp.float32),
        scratch_types=[
            pltpu.VMEM((ch,), jnp.int32),
            pltpu.VMEM((ch, d), jnp.float32),
            pltpu.SemaphoreType.DMA,
        ],
        compiler_params=pltpu.CompilerParams(use_tc_tiling_on_sc=False),
    )
    def k(table_hbm, idx_hbm, out_hbm, idx_v, rows_v, sem):
        wid = lax.axis_index("s") * _NC + lax.axis_index("c")

        def body(i, carry):
            base = wid * per_w + i * ch
            pltpu.sync_copy(idx_hbm.at[pl.ds(base, ch)], idx_v)
            pltpu.async_copy(table_hbm.at[idx_v], rows_v, sem).wait()
            pltpu.sync_copy(rows_v, out_hbm.at[pl.ds(base, ch)])
            return carry

        lax.fori_loop(0, n_ch, body, 0)

    return k(table, idx)


def _sc_scatter_add(vals, idx, zeros):
    e, d = vals.shape
    n = zeros.shape[0]
    per_w = e // _NW
    ch = 200
    n_ch = per_w // ch
    rows_pw = n // _NS
    mesh = plsc.VectorSubcoreMesh(core_axis_name="c", subcore_axis_name="s")

    @functools.partial(
        pl.kernel, mesh=mesh,
        out_type=jax.ShapeDtypeStruct((_NC, n, d), jnp.float32),
        scratch_types=[
            pltpu.VMEM((ch,), jnp.int32),
            pltpu.VMEM((ch, d), jnp.float32),
            pltpu.VMEM_SHARED((n, d), jnp.float32),
        ],
        compiler_params=pltpu.CompilerParams(use_tc_tiling_on_sc=False),
    )
    def k(vals_hbm, idx_hbm, zeros_hbm, out_hbm, idx_v, vals_v, acc_sh):
        cid = lax.axis_index("c")
        sid = lax.axis_index("s")
        wid = sid * _NC + cid
        pltpu.sync_copy(zeros_hbm.at[pl.ds(sid * rows_pw, rows_pw)],
                        acc_sh.at[pl.ds(sid * rows_pw, rows_pw)])
        plsc.subcore_barrier()

        def body(i, carry):
            base = wid * per_w + i * ch
            pltpu.sync_copy(idx_hbm.at[pl.ds(base, ch)], idx_v)
            pltpu.sync_copy(vals_hbm.at[pl.ds(base, ch)], vals_v)
            pltpu.sync_copy(vals_v, acc_sh.at[idx_v], add=True)
            return carry

        lax.fori_loop(0, n_ch, body, 0)
        plsc.subcore_barrier()
        pltpu.sync_copy(acc_sh.at[pl.ds(sid * rows_pw, rows_pw)],
                        out_hbm.at[cid, pl.ds(sid * rows_pw, rows_pw)])

    return k(vals, idx, zeros)


_BE = 8000
_BN = 2000


def _edge1_body(xs_ref, xd_ref, ea_ref, wl0_ref, wl1_ref, wr0_ref, wr1_ref,
                bb_ref, we_ref, attf_ref, s_ref, out_ref):
    x0 = xs_ref[:, 0:1]
    x1 = xs_ref[:, 1:2]
    y0 = xd_ref[:, 0:1]
    y1 = xd_ref[:, 1:2]
    e = (x0 * wl0_ref[0:1, :] + x1 * wl1_ref[0:1, :]
         + y0 * wr0_ref[0:1, :] + y1 * wr1_ref[0:1, :]
         + ea_ref[:, 0:1] * we_ref[0:1, :] + bb_ref[0:1, :])
    e = jnp.where(e >= 0.0, e, 0.2 * e)
    alpha = jnp.dot(e * attf_ref[0:1, :], s_ref[:, :],
                    preferred_element_type=jnp.float32)
    ae = jnp.exp(alpha)
    out_ref[:, :] = jnp.concatenate(
        [ae, ae * x0, ae * x1, jnp.zeros_like(ae)], axis=1)


def _node1_body(a1a_ref, a1b_ref, x_ref, r0_ref, r1_ref, rb_ref,
                wsk_ref, bb_ref, hpre_ref, st_ref):
    i = pl.program_id(0)
    a = a1a_ref[:, :] + a1b_ref[:, :]
    den = a[:, 0:8] + 1e-16
    g0 = a[:, 8:16] / den
    g1 = a[:, 16:24] / den
    s = a[:, 0:8] / den
    h1 = (jnp.dot(g0, r0_ref[:, :], preferred_element_type=jnp.float32)
          + jnp.dot(g1, r1_ref[:, :], preferred_element_type=jnp.float32)
          + jnp.dot(s, rb_ref[:, :], preferred_element_type=jnp.float32))
    h = (h1 + x_ref[:, 0:1] * wsk_ref[0:1, :]
         + x_ref[:, 1:2] * wsk_ref[1:2, :] + bb_ref[0:1, :])
    hpre_ref[:, :] = h

    @pl.when(i == 0)
    def _():
        st_ref[:, :] = jnp.zeros_like(st_ref)

    st_ref[0:1, :] += jnp.sum(h, axis=0, keepdims=True)
    st_ref[1:2, :] += jnp.sum(h * h, axis=0, keepdims=True)


def _node1b_body(hpre_ref, st_ref, gam_ref, bet_ref, wl2_ref, wr2_ref,
                 b2_ref, xl2_ref, xr2_ref):
    mu = st_ref[0:1, :] * (1.0 / _N)
    var = st_ref[1:2, :] * (1.0 / _N) - mu * mu
    h = (hpre_ref[:, :] - mu) * jax.lax.rsqrt(var + 1e-5)
    h = h * gam_ref[0:1, :] + bet_ref[0:1, :]
    h = jnp.where(h > 0.0, h, jnp.exp(h) - 1.0)
    xl2_ref[:, :] = jnp.dot(h, wl2_ref[:, :],
                            preferred_element_type=jnp.float32) + b2_ref[0:1, 0:16]
    xr2_ref[:, :] = jnp.dot(h, wr2_ref[:, :],
                            preferred_element_type=jnp.float32) + b2_ref[0:1, 16:32]


def _edge2_body(xls_ref, xrd_ref, ea_ref, we2_ref, att2_ref, out_ref):
    e = xls_ref[:, :] + xrd_ref[:, :] + ea_ref[:, 0:1] * we2_ref[0:1, :]
    e = jnp.where(e >= 0.0, e, 0.2 * e)
    alpha = jnp.sum(e * att2_ref[0:1, :], axis=1, keepdims=True)
    ae = jnp.exp(alpha)
    z = jnp.zeros_like(xls_ref[:, 1:16])
    out_ref[:, :] = jnp.concatenate([ae, z, ae * xls_ref[:, :]], axis=1)


def _node2_body(a2a_ref, a2b_ref, b2_ref, wt1_ref, bt1_ref, wc1_ref, bc1_ref,
                wt2_ref, bt2_ref, wc2_ref, bc2_ref, out_ref, dn2_ref):
    a = a2a_ref[:, :] + a2b_ref[:, :]
    dn2_ref[:, :] = a[:, 0:16]
    h2 = a[:, 16:32] / (a[:, 0:1] + 1e-16) + b2_ref[0:1, :]
    h2 = jnp.where(h2 > 0.0, h2, jnp.exp(h2) - 1.0)
    ht = jnp.dot(h2, wt1_ref[:, :], preferred_element_type=jnp.float32) + bt1_ref[0:1, :]
    hc = jnp.dot(h2, wc1_ref[:, :], preferred_element_type=jnp.float32) + bc1_ref[0:1, :]
    ht = jnp.dot(ht, wt2_ref[:, :], preferred_element_type=jnp.float32) + bt2_ref[0:1, :]
    hc = jnp.dot(hc, wc2_ref[:, :], preferred_element_type=jnp.float32) + bc2_ref[0:1, :]
    out_ref[:, :] = jnp.concatenate(
        [hc[:, 0:4], ht[:, 0:6], jnp.zeros_like(h2[:, 0:6])], axis=1)


def _a2_body(p2_ref, g2_ref, out_ref):
    a2 = p2_ref[:, 0:1] / (g2_ref[:, 0:1] + 1e-16)
    out_ref[:, :] = jnp.concatenate([a2] * 8, axis=1)


def _row_spec(b, w):
    return pl.BlockSpec((b, w), lambda i: (i, 0))


def _full_spec(shape):
    return pl.BlockSpec(shape, lambda i: tuple(0 for _ in shape))


def kernel(x, edge_index, edge_attr, Wl1, bl1, Wr1, br1, We1, att1, bias1,
           Wskip, bskip, gamma, beta, Wl2, bl2, Wr2, br2, We2, att2, bias2,
           Wc1, bc1, Wt1, bt1, Wc2, bc2, Wt2, bt2):
    src = edge_index[0]
    dst = edge_index[1]

    sel = jnp.repeat(jnp.eye(_H, dtype=jnp.float32), _D, axis=0)
    attf = att1.reshape(1, _HD)
    r0 = sel.T * Wl1[0][None, :]
    r1 = sel.T * Wl1[1][None, :]
    rb = sel.T * bl1[None, :]
    bb1 = (bl1 + br1).reshape(1, _HD)
    b2cat = jnp.concatenate([bl2, br2]).reshape(1, 32)

    x_pad = jnp.pad(x, ((0, 0), (0, 14)))
    zeros32 = jnp.zeros((_N, 32), jnp.float32)

    xs = _sc_gather(x_pad, src)
    xd = _sc_gather(x_pad, dst)

    ge = _E // _BE
    p1 = pl.pallas_call(
        _edge1_body,
        grid=(ge,),
        in_specs=[_row_spec(_BE, 16), _row_spec(_BE, 16), _row_spec(_BE, 1),
                  _full_spec((1, _HD)), _full_spec((1, _HD)),
                  _full_spec((1, _HD)), _full_spec((1, _HD)),
                  _full_spec((1, _HD)), _full_spec((1, _HD)),
                  _full_spec((1, _HD)), _full_spec((_HD, _H))],
        out_specs=_row_spec(_BE, 32),
        out_shape=jax.ShapeDtypeStruct((_E, 32), jnp.float32),
    )(xs, xd, edge_attr, Wl1[0].reshape(1, -1), Wl1[1].reshape(1, -1),
      Wr1[0].reshape(1, -1), Wr1[1].reshape(1, -1), bb1,
      We1.reshape(1, -1), attf, sel)

    a1 = _sc_scatter_add(p1, dst, zeros32)

    gn = _N // _BN
    hpre, stats = pl.pallas_call(
        _node1_body,
        grid=(gn,),
        in_specs=[_row_spec(_BN, 32), _row_spec(_BN, 32), _row_spec(_BN, 2),
                  _full_spec((_H, _HD)), _full_spec((_H, _HD)),
                  _full_spec((_H, _HD)), _full_spec((2, _HD)),
                  _full_spec((1, _HD))],
        out_specs=[_row_spec(_BN, _HD), _full_spec((2, _HD))],
        out_shape=[jax.ShapeDtypeStruct((_N, _HD), jnp.float32),
                   jax.ShapeDtypeStruct((2, _HD), jnp.float32)],
    )(a1[0], a1[1], x, r0, r1, rb, Wskip,
      (bias1 + bskip).reshape(1, -1))

    xl2, xr2 = pl.pallas_call(
        _node1b_body,
        grid=(gn,),
        in_specs=[_row_spec(_BN, _HD), _full_spec((2, _HD)),
                  _full_spec((1, _HD)), _full_spec((1, _HD)),
                  _full_spec((_HD, _D)), _full_spec((_HD, _D)),
                  _full_spec((1, 32))],
        out_specs=[_row_spec(_BN, _D), _row_spec(_BN, _D)],
        out_shape=[jax.ShapeDtypeStruct((_N, _D), jnp.float32),
                   jax.ShapeDtypeStruct((_N, _D), jnp.float32)],
    )(hpre, stats, gamma.reshape(1, -1), beta.reshape(1, -1), Wl2, Wr2, b2cat)

    xls = _sc_gather(xl2, src)
    xrd = _sc_gather(xr2, dst)

    p2 = pl.pallas_call(
        _edge2_body,
        grid=(ge,),
        in_specs=[_row_spec(_BE, 16), _row_spec(_BE, 16), _row_spec(_BE, 1),
                  _full_spec((1, _D)), _full_spec((1, _D))],
        out_specs=_row_spec(_BE, 32),
        out_shape=jax.ShapeDtypeStruct((_E, 32), jnp.float32),
    )(xls, xrd, edge_attr, We2.reshape(1, -1), att2.reshape(1, -1))

    a2t = _sc_scatter_add(p2, dst, zeros32)

    out16, dn2 = pl.pallas_call(
        _node2_body,
        grid=(gn,),
        in_specs=[_row_spec(_BN, 32), _row_spec(_BN, 32),
                  _full_spec((1, _D)),
                  _full_spec((_D, _D)), _full_spec((1, _D)),
                  _full_spec((_D, _D)), _full_spec((1, _D)),
                  _full_spec((_D, 6)), _full_spec((1, 6)),
                  _full_spec((_D, 4)), _full_spec((1, 4))],
        out_specs=[_row_spec(_BN, 16), _row_spec(_BN, 16)],
        out_shape=[jax.ShapeDtypeStruct((_N, 16), jnp.float32),
                   jax.ShapeDtypeStruct((_N, 16), jnp.float32)],
    )(a2t[0], a2t[1], bias2.reshape(1, -1), Wt1, bt1.reshape(1, -1),
      Wc1, bc1.reshape(1, -1), Wt2, bt2.reshape(1, -1), Wc2, bc2.reshape(1, -1))

    g2 = _sc_gather(dn2, dst)

    a2v = pl.pallas_call(
        _a2_body,
        grid=(ge,),
        in_specs=[_row_spec(_BE, 32), _row_spec(_BE, 16)],
        out_specs=_row_spec(_BE, 8),
        out_shape=jax.ShapeDtypeStruct((_E, 8), jnp.float32),
    )(p2, g2)

    return out16[:, 0:10], a2v[:, 0:1]
